# baseline (speedup 1.0000x reference)
import jax
import jax.numpy as jnp
from jax import lax
from jax.experimental import pallas as pl
from jax.experimental.pallas import tpu as pltpu
from jax.experimental.pallas import tpu_sc as plsc

BATCH = 4
MAX_LEN = 2048
EMBED = 1024
B_SC = 2
TC_B = BATCH - B_SC
NC, NS, L = 2, 16, 16
NW = NC * NS
T_PER_W = MAX_LEN // NW
CT = 8
NCHUNK = T_PER_W // CT
NBUF = 3
VREGS_PER_ROW = EMBED // L
G = 8


def _sc_body(seq_hbm, tok_hbm, pos_hbm, out_hbm, idx_v,
             rows0, rows1, rows2, pos0, pos1, pos2,
             gsem0, gsem1, gsem2, ssem0, ssem1, ssem2):
    wid = lax.axis_index("s") * NC + lax.axis_index("c")
    tw0 = wid * T_PER_W
    for b in range(B_SC):
        pltpu.sync_copy(seq_hbm.at[b, pl.ds(tw0, T_PER_W)], idx_v.at[b])

    rows = [rows0, rows1, rows2]
    pos = [pos0, pos1, pos2]
    gsem = [gsem0, gsem1, gsem2]
    ssem = [ssem0, ssem1, ssem2]

    def start_unit(c):
        buf = c % NBUF
        t0 = tw0 + c * CT
        descs = [pltpu.async_copy(pos_hbm.at[pl.ds(t0, CT)], pos[buf], gsem[buf])]
        for b in range(B_SC):
            descs.append(pltpu.async_copy(
                tok_hbm.at[idx_v.at[b, pl.ds(c * CT, CT)]],
                rows[buf].at[b], gsem[buf]))
        return descs

    pend_g = {c: start_unit(c) for c in range(NBUF - 1)}
    pend_s = {}
    for c in range(NCHUNK):
        buf = c % NBUF
        nxt = c + NBUF - 1
        if nxt < NCHUNK:
            for d in pend_s.pop(nxt % NBUF, ()):
                d.wait()
            pend_g[nxt] = start_unit(nxt)
        for d in pend_g.pop(c):
            d.wait()

        def add_j(j, carry, _buf=buf):
            sl = pl.ds(j * L, L)
            for r in range(CT):
                p = pos[_buf][r, sl]
                for b in range(B_SC):
                    rows[_buf][b, r, sl] = rows[_buf][b, r, sl] + p
            return carry

        lax.fori_loop(0, VREGS_PER_ROW, add_j, 0)

        t0 = tw0 + c * CT
        pend_s[buf] = [
            pltpu.async_copy(rows[buf].at[b], out_hbm.at[b, pl.ds(t0, CT)],
                             ssem[buf])
            for b in range(B_SC)
        ]
    for descs in pend_s.values():
        for d in descs:
            d.wait()


def _sc_part(seq_sc, token_weight, position_weight):
    mesh = plsc.VectorSubcoreMesh(core_axis_name="c", subcore_axis_name="s")
    f = pl.kernel(
        _sc_body,
        out_type=jax.ShapeDtypeStruct((B_SC, MAX_LEN, EMBED), jnp.float32),
        mesh=mesh,
        scratch_types=(
            [pltpu.VMEM((B_SC, T_PER_W), jnp.int32)]
            + [pltpu.VMEM((B_SC, CT, EMBED), jnp.float32)] * NBUF
            + [pltpu.VMEM((CT, EMBED), jnp.float32)] * NBUF
            + [pltpu.SemaphoreType.DMA] * (2 * NBUF)
        ),
    )
    return f(seq_sc, token_weight, position_weight)


def _tc_body(idx_ref, *refs):
    tok_blks = refs[:TC_B * G]
    pos_blk = refs[TC_B * G]
    out_blk = refs[TC_B * G + 1]
    p = pos_blk[...]
    for b in range(TC_B):
        rows = jnp.concatenate(
            [tok_blks[b * G + j][...].reshape(1, EMBED) for j in range(G)],
            axis=0)
        out_blk[b, :, :] = rows + p


def _tc_part(seq_tc, token_weight, position_weight):
    steps = MAX_LEN // G
    idx_flat = seq_tc.reshape(-1)

    def tok_map(i, idx_ref, b=0, j=0):
        return (idx_ref[b * MAX_LEN + G * i + j], 0)

    in_specs = [
        pl.BlockSpec((1, 1, EMBED),
                     (lambda i, idx_ref, b=b, j=j:
                      (idx_ref[b * MAX_LEN + G * i + j], 0, 0)))
        for b in range(TC_B) for j in range(G)
    ]
    tok3 = token_weight.reshape(token_weight.shape[0], 1, EMBED)
    in_specs.append(pl.BlockSpec((G, EMBED), lambda i, idx_ref: (i, 0)))
    grid_spec = pltpu.PrefetchScalarGridSpec(
        num_scalar_prefetch=1,
        grid=(steps,),
        in_specs=in_specs,
        out_specs=pl.BlockSpec((TC_B, G, EMBED), lambda i, idx_ref: (0, i, 0)),
    )
    return pl.pallas_call(
        _tc_body,
        grid_spec=grid_spec,
        out_shape=jax.ShapeDtypeStruct((TC_B, MAX_LEN, EMBED), jnp.float32),
    )(idx_flat, *([tok3] * (TC_B * G)), position_weight)


def kernel(sequence, token_weight, position_weight):
    sc_out = _sc_part(sequence[:B_SC], token_weight, position_weight)
    tc_out = _tc_part(sequence[B_SC:], token_weight, position_weight)
    return jnp.concatenate([sc_out, tc_out], axis=0)


# speedup vs baseline: 8.0719x; 8.0719x over previous
import jax
import jax.numpy as jnp
from jax import lax
from jax.experimental import pallas as pl
from jax.experimental.pallas import tpu as pltpu
from jax.experimental.pallas import tpu_sc as plsc

BATCH = 4
MAX_LEN = 2048
EMBED = 1024
NC, NS, L = 2, 16, 16
NW = NC * NS
T_PER_W = MAX_LEN // NW
CT = 16
NQ = T_PER_W // CT
NPAIR = 2
NU = NQ * NPAIR
RPU = NPAIR * CT
NBUF = 3
VREGS_PER_ROW = EMBED // L


def _body(seq_hbm, tok_hbm, pos_hbm, out_hbm, idx_v,
          rows0, rows1, rows2, pos_v,
          gsem0, gsem1, gsem2, psem, ssem0, ssem1, ssem2, isem):
    wid = lax.axis_index("s") * NC + lax.axis_index("c")
    tw0 = wid * T_PER_W
    idescs = []
    for q in range(NQ):
        for pr in range(NPAIR):
            u = q * NPAIR + pr
            for i in range(2):
                idescs.append(pltpu.async_copy(
                    seq_hbm.at[2 * pr + i, pl.ds(tw0 + q * CT, CT)],
                    idx_v.at[u, pl.ds(i * CT, CT)], isem))
    for d in idescs:
        d.wait()

    rows = [rows0, rows1, rows2]
    gsem = [gsem0, gsem1, gsem2]
    ssem = [ssem0, ssem1, ssem2]

    def start_pos(q):
        return [pltpu.async_copy(pos_hbm.at[pl.ds(tw0 + q * CT, CT)],
                                 pos_v, psem)]

    def start_unit(u):
        rb = u % NBUF
        return [pltpu.async_copy(tok_hbm.at[idx_v.at[u]], rows[rb], gsem[rb])]

    pend_pos = {0: start_pos(0)}
    pend_g = {0: start_unit(0), 1: start_unit(1)}
    pend_s = {}
    for u in range(NU):
        q, pr = divmod(u, NPAIR)
        rb = u % NBUF
        nxt = u + NBUF - 1
        if nxt < NU:
            for d in pend_s.pop(nxt % NBUF, ()):
                d.wait()
            pend_g[nxt] = start_unit(nxt)
        for d in pend_g.pop(u):
            d.wait()
        if pr == 0:
            for d in pend_pos.pop(q):
                d.wait()

        def add_j(j, carry, _rb=rb):
            sl = pl.ds(j * L, L)
            for r in range(CT):
                p = pos_v[r, sl]
                for i in range(2):
                    rows[_rb][i * CT + r, sl] = rows[_rb][i * CT + r, sl] + p
            return carry

        lax.fori_loop(0, VREGS_PER_ROW, add_j, 0)

        if pr == NPAIR - 1 and q + 1 < NQ:
            pend_pos[q + 1] = start_pos(q + 1)

        t0 = tw0 + q * CT
        pend_s[rb] = [
            pltpu.async_copy(rows[rb].at[pl.ds(i * CT, CT)],
                             out_hbm.at[2 * pr + i, pl.ds(t0, CT)], ssem[rb])
            for i in range(2)
        ]
    for descs in pend_s.values():
        for d in descs:
            d.wait()


def kernel(sequence, token_weight, position_weight):
    mesh = plsc.VectorSubcoreMesh(core_axis_name="c", subcore_axis_name="s")
    f = pl.kernel(
        _body,
        out_type=jax.ShapeDtypeStruct((BATCH, MAX_LEN, EMBED), jnp.float32),
        mesh=mesh,
        scratch_types=(
            [pltpu.VMEM((NU, RPU), jnp.int32)]
            + [pltpu.VMEM((RPU, EMBED), jnp.float32)] * NBUF
            + [pltpu.VMEM((CT, EMBED), jnp.float32)]
            + [pltpu.SemaphoreType.DMA] * 8
        ),
    )
    return f(sequence, token_weight, position_weight)


# speedup vs baseline: 9.5072x; 1.1778x over previous
import jax
import jax.numpy as jnp
from jax import lax
from jax.experimental import pallas as pl
from jax.experimental.pallas import tpu as pltpu
from jax.experimental.pallas import tpu_sc as plsc

BATCH = 4
MAX_LEN = 2048
EMBED = 1024
NC, NS, L = 2, 16, 16
NW = NC * NS
T_PER_W = MAX_LEN // NW
CT = 8
NCHUNK = T_PER_W // CT
NBUF = 3
VREGS_PER_ROW = EMBED // L


def _body(seq_hbm, tok_hbm, pos_hbm, out_hbm, idx_v,
          rows0, rows1, rows2, pos0, pos1, pos2,
          gsem0, gsem1, gsem2, ssem0, ssem1, ssem2):
    wid = lax.axis_index("s") * NC + lax.axis_index("c")
    tw0 = wid * T_PER_W
    for b in range(BATCH):
        pltpu.sync_copy(seq_hbm.at[b, pl.ds(tw0, T_PER_W)], idx_v.at[b])

    rows = [rows0, rows1, rows2]
    pos = [pos0, pos1, pos2]
    gsem = [gsem0, gsem1, gsem2]
    ssem = [ssem0, ssem1, ssem2]

    def start_unit(c):
        buf = c % NBUF
        t0 = tw0 + c * CT
        descs = [pltpu.async_copy(pos_hbm.at[pl.ds(t0, CT)], pos[buf], gsem[buf])]
        for b in range(BATCH):
            descs.append(pltpu.async_copy(
                tok_hbm.at[idx_v.at[b, pl.ds(c * CT, CT)]],
                rows[buf].at[b], gsem[buf]))
        return descs

    pend_g = {c: start_unit(c) for c in range(NBUF - 1)}
    pend_s = {}
    for c in range(NCHUNK):
        buf = c % NBUF
        nxt = c + NBUF - 1
        if nxt < NCHUNK:
            for d in pend_s.pop(nxt % NBUF, ()):
                d.wait()
            pend_g[nxt] = start_unit(nxt)
        for d in pend_g.pop(c):
            d.wait()

        t0 = tw0 + c * CT
        sdescs = []
        for pr in range(BATCH // 2):

            def add_j(j, carry, _buf=buf, _pr=pr):
                sl = pl.ds(j * L, L)
                for r in range(CT):
                    p = pos[_buf][r, sl]
                    for b in (2 * _pr, 2 * _pr + 1):
                        rows[_buf][b, r, sl] = rows[_buf][b, r, sl] + p
                return carry

            lax.fori_loop(0, VREGS_PER_ROW, add_j, 0)
            for b in (2 * pr, 2 * pr + 1):
                sdescs.append(pltpu.async_copy(
                    rows[buf].at[b], out_hbm.at[b, pl.ds(t0, CT)], ssem[buf]))
        pend_s[buf] = sdescs
    for descs in pend_s.values():
        for d in descs:
            d.wait()


def kernel(sequence, token_weight, position_weight):
    mesh = plsc.VectorSubcoreMesh(core_axis_name="c", subcore_axis_name="s")
    f = pl.kernel(
        _body,
        out_type=jax.ShapeDtypeStruct((BATCH, MAX_LEN, EMBED), jnp.float32),
        mesh=mesh,
        scratch_types=[
            pltpu.VMEM((BATCH, T_PER_W), jnp.int32),
            pltpu.VMEM((BATCH, CT, EMBED), jnp.float32),
            pltpu.VMEM((BATCH, CT, EMBED), jnp.float32),
            pltpu.VMEM((BATCH, CT, EMBED), jnp.float32),
            pltpu.VMEM((CT, EMBED), jnp.float32),
            pltpu.VMEM((CT, EMBED), jnp.float32),
            pltpu.VMEM((CT, EMBED), jnp.float32),
            pltpu.SemaphoreType.DMA,
            pltpu.SemaphoreType.DMA,
            pltpu.SemaphoreType.DMA,
            pltpu.SemaphoreType.DMA,
            pltpu.SemaphoreType.DMA,
            pltpu.SemaphoreType.DMA,
        ],
    )
    return f(sequence, token_weight, position_weight)
